# baseline (speedup 1.0000x reference)
import functools

import jax
import jax.numpy as jnp
from jax import lax
from jax.experimental import pallas as pl
from jax.experimental.pallas import tpu as pltpu
from jax.experimental.pallas import tpu_sc as plsc

N = 10000
IN_DIM = 128
F = 64
M = 64
L = 3
EIN = 2 * F + 1
CUTOFF = 0.09
E_MAX = 700000

NC, NS = 2, 16
NW = NC * NS

NP = 10240
RPW = 10016 // NW
CAPW = 12288
E_SLAB = NW * CAPW
EPW = CAPW
SG = 512
NSG = EPW // SG
CW = 16
N_TAB = N + 16
N_ACC = N + 16

EDGE_BLK = 4096
TW = 72

_mesh = plsc.VectorSubcoreMesh(core_axis_name="c", subcore_axis_name="s")


def _silu(x):
    return x * jax.nn.sigmoid(x)


def _mask_body(cb_ref, ct_ref, r2r_ref, r2c_ref, mask_ref):
    i = pl.program_id(0)
    d2 = (r2r_ref[...] + r2c_ref[...]
          - 2.0 * jnp.dot(cb_ref[...], ct_ref[...],
                          preferred_element_type=jnp.float32))
    rows = i * 128 + jax.lax.broadcasted_iota(jnp.int32, (128, NP), 0)
    cols = jax.lax.broadcasted_iota(jnp.int32, (128, NP), 1)
    m = (d2 <= CUTOFF * CUTOFF) & (rows != cols)
    mask_ref[...] = m.astype(jnp.int32)


def _mask_kernel(c8, c8t, r2r, r2c):
    grid = (NP // 128,)
    return pl.pallas_call(
        _mask_body,
        grid=grid,
        in_specs=[
            pl.BlockSpec((128, 8), lambda i: (i, 0)),
            pl.BlockSpec((8, NP), lambda i: (0, 0)),
            pl.BlockSpec((128, 1), lambda i: (i, 0)),
            pl.BlockSpec((1, NP), lambda i: (0, 0)),
        ],
        out_specs=pl.BlockSpec((128, NP), lambda i: (i, 0)),
        out_shape=jax.ShapeDtypeStruct((NP, NP), jnp.int32),
    )(c8, c8t, r2r, r2c)


@functools.partial(
    pl.kernel,
    out_type=[
        jax.ShapeDtypeStruct((NW, CAPW), jnp.int32),
        jax.ShapeDtypeStruct((NW, CAPW), jnp.int32),
        jax.ShapeDtypeStruct((NW, 16), jnp.int32),
    ],
    mesh=_mesh,
    scratch_types=[
        pltpu.VMEM((NP,), jnp.int32),
        pltpu.VMEM((NP,), jnp.int32),
        pltpu.VMEM((16,), jnp.int32),
        pltpu.VMEM((CAPW + 16,), jnp.int32),
        pltpu.VMEM((CAPW + 16,), jnp.int32),
        pltpu.SemaphoreType.DMA,
    ],
    compiler_params=pltpu.CompilerParams(use_tc_tiling_on_sc=False,
                                         needs_layout_passes=False),
)
def _sc_compact(mask_hbm, src_out, dst_out, cnt_out,
                mb0, mb1, cbuf, st_s, st_d, sem_m):
    cid = lax.axis_index("c")
    sid = lax.axis_index("s")
    wid = sid * NC + cid
    iota = jnp.arange(16, dtype=jnp.int32)
    zeros16 = jnp.zeros((16,), jnp.int32)
    nfill16 = jnp.full((16,), N, jnp.int32)
    rbase = wid * RPW

    def init_body(g, carry):
        st_s[pl.ds(g * 16, 16)] = zeros16
        st_d[pl.ds(g * 16, 16)] = nfill16
        return carry

    lax.fori_loop(0, (CAPW + 16) // 16, init_body, 0)

    def scan_row(mbuf, row, fill):
        rowvec = jnp.full((16,), 1, jnp.int32) * row

        def one(g, f):
            mvec = mbuf[pl.ds(g * 16, 16)]
            msk = mvec > 0
            jvec = iota + g * 16
            plsc.store_compressed(st_s.at[pl.ds(f, 16)], jvec, mask=msk)
            plsc.store_compressed(st_d.at[pl.ds(f, 16)], rowvec, mask=msk)
            cnt = plsc.all_reduce_population_count(msk)[0]
            return jnp.minimum(f + cnt, CAPW)

        def group_body(g2, f):
            f = one(g2 * 4, f)
            f = one(g2 * 4 + 1, f)
            f = one(g2 * 4 + 2, f)
            return one(g2 * 4 + 3, f)

        return lax.fori_loop(0, NP // 64, group_body, fill)

    def fetch(row, mbuf):
        return pltpu.async_copy(mask_hbm.at[pl.ds(row * NP, NP)], mbuf, sem_m)

    fetch(rbase, mb0)

    def pair_body(rr, fill):
        r0 = rbase + rr * 2
        pltpu.make_async_copy(mask_hbm.at[pl.ds(r0 * NP, NP)], mb0, sem_m).wait()
        fetch(r0 + 1, mb1)
        fill = scan_row(mb0, r0, fill)
        pltpu.make_async_copy(mask_hbm.at[pl.ds((r0 + 1) * NP, NP)], mb1,
                              sem_m).wait()
        fetch(r0 + 2, mb0)
        return scan_row(mb1, r0 + 1, fill)

    fill = lax.fori_loop(0, (RPW - 1) // 2, pair_body, jnp.int32(0))
    rlast = rbase + RPW - 1
    pltpu.make_async_copy(mask_hbm.at[pl.ds(rlast * NP, NP)], mb0, sem_m).wait()
    fill = scan_row(mb0, rlast, fill)
    pltpu.sync_copy(st_s.at[pl.ds(0, CAPW)], src_out.at[wid])
    pltpu.sync_copy(st_d.at[pl.ds(0, CAPW)], dst_out.at[wid])
    cbuf[...] = jnp.full((16,), 1, jnp.int32) * fill
    pltpu.sync_copy(cbuf, cnt_out.at[wid])


@functools.partial(
    pl.kernel,
    out_type=[
        jax.ShapeDtypeStruct((E_SLAB, TW), jnp.float32),
        jax.ShapeDtypeStruct((E_SLAB, TW), jnp.float32),
    ],
    mesh=_mesh,
    scratch_types=[
        pltpu.VMEM((4, 128), jnp.int32),
        pltpu.VMEM((4, 128), jnp.int32),
        pltpu.VMEM((SG, TW), jnp.float32),
        pltpu.VMEM((SG, TW), jnp.float32),
        pltpu.VMEM((16,), jnp.int32),
        pltpu.VMEM_SHARED((N_TAB, TW), jnp.float32),
        pltpu.SemaphoreType.DMA,
        pltpu.SemaphoreType.DMA,
        pltpu.SemaphoreType.DMA,
    ],
    compiler_params=pltpu.CompilerParams(use_tc_tiling_on_sc=False),
)
def _sc_gather(tab_hbm, src2d, dst2d, cnt_hbm,
               gd_hbm, gs_hbm,
               sidx, didx, gd_b, gs_b, ccv, sh_tab,
               sem_i, sem_g, sem_w):
    cid = lax.axis_index("c")
    sid = lax.axis_index("s")
    wid = sid * NC + cid
    pltpu.sync_copy(cnt_hbm.at[wid], ccv)
    trows = N_TAB // NS
    tb = sid * trows
    pltpu.sync_copy(tab_hbm.at[pl.ds(tb, trows)], sh_tab.at[pl.ds(tb, trows)])
    plsc.subcore_barrier()

    def body(g, carry):
        ebase = wid * EPW + g * SG
        rbase = wid * (EPW // 128) + g * (SG // 128)
        h1 = pltpu.async_copy(src2d.at[pl.ds(rbase, 4)], sidx, sem_i)
        h2 = pltpu.async_copy(dst2d.at[pl.ds(rbase, 4)], didx, sem_i)
        h1.wait()
        h2.wait()
        hs = []
        for j in range(4):
            sl = pl.ds(j * 128, 128)
            hs.append(pltpu.async_copy(sh_tab.at[sidx.at[j]], gs_b.at[sl], sem_g))
            hs.append(pltpu.async_copy(sh_tab.at[didx.at[j]], gd_b.at[sl], sem_g))
        for hh in hs:
            hh.wait()
        w = [pltpu.async_copy(gd_b, gd_hbm.at[pl.ds(ebase, SG)], sem_w),
             pltpu.async_copy(gs_b, gs_hbm.at[pl.ds(ebase, SG)], sem_w)]
        for hh in w:
            hh.wait()
        return carry

    nch = (ccv[...][0] + (SG - 1)) // SG
    lax.fori_loop(0, nch, body, 0)


@functools.partial(
    pl.kernel,
    out_type=[
        jax.ShapeDtypeStruct((NC, N_ACC, F), jnp.float32),
        jax.ShapeDtypeStruct((NC, N_ACC, CW), jnp.float32),
    ],
    mesh=_mesh,
    scratch_types=[
        pltpu.VMEM((4, 128), jnp.int32),
        pltpu.VMEM((SG, F), jnp.float32),
        pltpu.VMEM((SG, CW), jnp.float32),
        pltpu.VMEM((16,), jnp.int32),
        pltpu.VMEM_SHARED((N_ACC, F), jnp.float32),
        pltpu.VMEM_SHARED((N_ACC, CW), jnp.float32),
        pltpu.SemaphoreType.DMA,
        pltpu.SemaphoreType.DMA,
    ],
    compiler_params=pltpu.CompilerParams(use_tc_tiling_on_sc=False),
)
def _sc_scatter(m_hbm, wrel_hbm, dstm2d, cnt_hbm, zh_hbm, zc_hbm,
                out_h, out_c,
                didx, m_b, w_b, ccv, acc_h, acc_c, sem_i, sem_s):
    cid = lax.axis_index("c")
    sid = lax.axis_index("s")
    wid = sid * NC + cid
    pltpu.sync_copy(cnt_hbm.at[wid], ccv)
    rows_per_sub = N_ACC // NS
    rb = sid * rows_per_sub
    pltpu.sync_copy(zh_hbm.at[pl.ds(rb, rows_per_sub)],
                    acc_h.at[pl.ds(rb, rows_per_sub)])
    pltpu.sync_copy(zc_hbm.at[pl.ds(rb, rows_per_sub)],
                    acc_c.at[pl.ds(rb, rows_per_sub)])
    plsc.subcore_barrier()

    def body(g, carry):
        ebase = wid * EPW + g * SG
        rbase = wid * (EPW // 128) + g * (SG // 128)
        pltpu.async_copy(dstm2d.at[pl.ds(rbase, 4)], didx, sem_i).wait()
        a = pltpu.async_copy(m_hbm.at[pl.ds(ebase, SG)], m_b, sem_i)
        b = pltpu.async_copy(wrel_hbm.at[pl.ds(ebase, SG)], w_b, sem_i)
        a.wait()
        b.wait()
        hs = []
        for j in range(4):
            sl = pl.ds(j * 128, 128)
            hs.append(pltpu.async_copy(m_b.at[sl], acc_h.at[didx.at[j]],
                                       sem_s, add=True))
            hs.append(pltpu.async_copy(w_b.at[sl], acc_c.at[didx.at[j]],
                                       sem_s, add=True))
        for hh in hs:
            hh.wait()
        return carry

    nch = (ccv[...][0] + (SG - 1)) // SG
    lax.fori_loop(0, nch, body, 0)
    plsc.subcore_barrier()
    pltpu.sync_copy(acc_h.at[pl.ds(rb, rows_per_sub)],
                    out_h.at[cid].at[pl.ds(rb, rows_per_sub)])
    pltpu.sync_copy(acc_c.at[pl.ds(rb, rows_per_sub)],
                    out_c.at[cid].at[pl.ds(rb, rows_per_sub)])


BPW = CAPW // EDGE_BLK


def _edge_mlp_body(cnt_ref, gd_ref, gs_ref,
                   wi_ref, wj_ref, wr_ref, b1_ref,
                   w2_ref, b2_ref, cw1_ref, cb1_ref, cw2_ref, cb2_ref,
                   cns_ref,
                   m_ref, wrel_ref):
    wkr = pl.program_id(0)
    j = pl.program_id(1)
    jm = jnp.maximum((cnt_ref[wkr] + EDGE_BLK - 1) // EDGE_BLK - 1, 0)

    @pl.when(j <= jm)
    def _():
        gd = gd_ref[...]
        gs = gs_ref[...]
        xi = gd[:, :F]
        xj = gs[:, :F]
        rel = jnp.pad(gs[:, F:TW] - gd[:, F:TW], ((0, 0), (0, CW - (TW - F))))
        rd = jnp.sum(rel * rel, axis=-1, keepdims=True)
        pre = (jnp.dot(xi, wi_ref[...], preferred_element_type=jnp.float32)
               + jnp.dot(xj, wj_ref[...], preferred_element_type=jnp.float32)
               + rd * wr_ref[...] + b1_ref[...])
        m1 = _silu(pre)
        m = _silu(jnp.dot(m1, w2_ref[...], preferred_element_type=jnp.float32)
                  + b2_ref[...])
        t = _silu(jnp.dot(m, cw1_ref[...], preferred_element_type=jnp.float32)
                  + cb1_ref[...])
        w = (jnp.dot(t, cw2_ref[...], preferred_element_type=jnp.float32)
             + cb2_ref[...])
        nrm = jnp.sqrt(rd)
        rel_n = rel / jnp.maximum(nrm, 1e-8) * cns_ref[0, 0]
        m_ref[...] = m
        wrel_ref[...] = w * rel_n


def _edge_mlp(gd, gs, cnts, wi, wj, wr, b1, w2, b2, cw1, cb1, cw2, cb2, cns):
    def eb(w, j, cref):
        jm = jnp.maximum((cref[w] + EDGE_BLK - 1) // EDGE_BLK - 1, 0)
        return (w * BPW + jnp.minimum(j, jm), 0)

    full = lambda w, j, cref: (0, 0)
    grid_spec = pltpu.PrefetchScalarGridSpec(
        num_scalar_prefetch=1,
        grid=(NW, BPW),
        in_specs=[
            pl.BlockSpec((EDGE_BLK, TW), eb),
            pl.BlockSpec((EDGE_BLK, TW), eb),
            pl.BlockSpec((F, 2 * EIN), full),
            pl.BlockSpec((F, 2 * EIN), full),
            pl.BlockSpec((1, 2 * EIN), full),
            pl.BlockSpec((1, 2 * EIN), full),
            pl.BlockSpec((2 * EIN, M), full),
            pl.BlockSpec((1, M), full),
            pl.BlockSpec((M, 4 * M), full),
            pl.BlockSpec((1, 4 * M), full),
            pl.BlockSpec((4 * M, 1), full),
            pl.BlockSpec((1, 1), full),
            pl.BlockSpec((1, 1), full),
        ],
        out_specs=[
            pl.BlockSpec((EDGE_BLK, M), eb),
            pl.BlockSpec((EDGE_BLK, CW), eb),
        ],
    )
    return pl.pallas_call(
        _edge_mlp_body,
        grid_spec=grid_spec,
        out_shape=[
            jax.ShapeDtypeStruct((E_SLAB, M), jnp.float32),
            jax.ShapeDtypeStruct((E_SLAB, CW), jnp.float32),
        ],
    )(cnts, gd, gs, wi, wj, wr, b1, w2, b2, cw1, cb1, cw2, cb2, cns)


def kernel(coors, feats, emb_w, emb_b, ew1, eb1, ew2, eb2, nnw, nnb,
           nw1, nb1, nw2, nb2, cw1, cb1, cw2, cb2, cns, ln_w, ln_b):
    n = coors.shape[0]
    c8 = jnp.zeros((NP, 8), jnp.float32).at[:n, :3].set(coors)
    r2 = jnp.full((NP,), 1e9, jnp.float32).at[:n].set(
        jnp.sum(coors * coors, axis=1))
    mask = _mask_kernel(c8, c8.T, r2[:, None], r2[None, :])
    src_slab, dst_slab, cnt = _sc_compact(mask.reshape(NP * NP))
    src2d = src_slab.reshape(E_SLAB // 128, 128)
    dst2d = dst_slab.reshape(E_SLAB // 128, 128)

    zh = jnp.zeros((N_ACC, F), jnp.float32)
    zc = jnp.zeros((N_ACC, CW), jnp.float32)

    h = feats @ emb_w + emb_b
    c = coors
    for l in range(L):
        tab = jnp.zeros((N_TAB, TW), jnp.float32)
        tab = tab.at[:n, :F].set(h).at[:n, F:F + 3].set(c)
        gd, gs = _sc_gather(tab, src2d, dst2d, cnt)
        m, wrel = _edge_mlp(
            gd, gs, cnt[:, 0],
            ew1[l][:F], ew1[l][F:2 * F], ew1[l][2 * F:2 * F + 1],
            eb1[l][None, :], ew2[l], eb2[l][None, :],
            cw1[l], cb1[l][None, :], cw2[l], cb2[l][None, :],
            cns[l][None, :],
        )
        hacc, cacc = _sc_scatter(m, wrel, dst2d, cnt, zh, zc)
        m_i = hacc[0, :N] + hacc[1, :N]
        c = c + (cacc[0, :N, :3] + cacc[1, :N, :3])
        mu = jnp.mean(h)
        sd = jnp.std(h)
        hf = (h - mu) / (sd + 1e-5) * nnw[l] + nnb[l]
        ho = _silu(jnp.concatenate([hf, m_i], axis=-1) @ nw1[l] + nb1[l]) @ nw2[l] + nb2[l]
        z = 2.0 * h + ho
        zm = jnp.mean(z, axis=-1, keepdims=True)
        zv = jnp.var(z, axis=-1, keepdims=True)
        h = (z - zm) / jnp.sqrt(zv + 1e-5) * ln_w + ln_b
    return c, h


# speedup vs baseline: 1.0175x; 1.0175x over previous
import functools

import jax
import jax.numpy as jnp
from jax import lax
from jax.experimental import pallas as pl
from jax.experimental.pallas import tpu as pltpu
from jax.experimental.pallas import tpu_sc as plsc

N = 10000
IN_DIM = 128
F = 64
M = 64
L = 3
EIN = 2 * F + 1
CUTOFF = 0.09
E_MAX = 700000

NC, NS = 2, 16
NW = NC * NS

NP = 10240
RPW = 10016 // NW
CAPW = 12288
E_SLAB = NW * CAPW
EPW = CAPW
SG = 512
NSG = EPW // SG
CW = 16
N_TAB = N + 16
N_ACC = N + 16

EDGE_BLK = 2048
TW = 72

_mesh = plsc.VectorSubcoreMesh(core_axis_name="c", subcore_axis_name="s")


def _silu(x):
    return x * jax.nn.sigmoid(x)


def _mask_body(cb_ref, ct_ref, r2r_ref, r2c_ref, mask_ref):
    i = pl.program_id(0)
    d2 = (r2r_ref[...] + r2c_ref[...]
          - 2.0 * jnp.dot(cb_ref[...], ct_ref[...],
                          preferred_element_type=jnp.float32))
    rows = i * 128 + jax.lax.broadcasted_iota(jnp.int32, (128, NP), 0)
    cols = jax.lax.broadcasted_iota(jnp.int32, (128, NP), 1)
    m = (d2 <= CUTOFF * CUTOFF) & (rows != cols)
    mask_ref[...] = m.astype(jnp.int32)


def _mask_kernel(c8, c8t, r2r, r2c):
    grid = (NP // 128,)
    return pl.pallas_call(
        _mask_body,
        grid=grid,
        in_specs=[
            pl.BlockSpec((128, 8), lambda i: (i, 0)),
            pl.BlockSpec((8, NP), lambda i: (0, 0)),
            pl.BlockSpec((128, 1), lambda i: (i, 0)),
            pl.BlockSpec((1, NP), lambda i: (0, 0)),
        ],
        out_specs=pl.BlockSpec((128, NP), lambda i: (i, 0)),
        out_shape=jax.ShapeDtypeStruct((NP, NP), jnp.int32),
    )(c8, c8t, r2r, r2c)


@functools.partial(
    pl.kernel,
    out_type=[
        jax.ShapeDtypeStruct((NW, CAPW), jnp.int32),
        jax.ShapeDtypeStruct((NW, CAPW), jnp.int32),
        jax.ShapeDtypeStruct((NW, 16), jnp.int32),
    ],
    mesh=_mesh,
    scratch_types=[
        pltpu.VMEM((NP,), jnp.int32),
        pltpu.VMEM((NP,), jnp.int32),
        pltpu.VMEM((16,), jnp.int32),
        pltpu.VMEM((CAPW + 16,), jnp.int32),
        pltpu.VMEM((CAPW + 16,), jnp.int32),
        pltpu.SemaphoreType.DMA,
    ],
    compiler_params=pltpu.CompilerParams(use_tc_tiling_on_sc=False,
                                         needs_layout_passes=False),
)
def _sc_compact(mask_hbm, src_out, dst_out, cnt_out,
                mb0, mb1, cbuf, st_s, st_d, sem_m):
    cid = lax.axis_index("c")
    sid = lax.axis_index("s")
    wid = sid * NC + cid
    iota = jnp.arange(16, dtype=jnp.int32)
    zeros16 = jnp.zeros((16,), jnp.int32)
    nfill16 = jnp.full((16,), N, jnp.int32)
    rbase = wid * RPW

    def init_body(g, carry):
        st_s[pl.ds(g * 16, 16)] = zeros16
        st_d[pl.ds(g * 16, 16)] = nfill16
        return carry

    lax.fori_loop(0, (CAPW + 16) // 16, init_body, 0)

    def scan_row(mbuf, row, fill):
        rowvec = jnp.full((16,), 1, jnp.int32) * row

        def one(g, f):
            mvec = mbuf[pl.ds(g * 16, 16)]
            msk = mvec > 0
            jvec = iota + g * 16
            plsc.store_compressed(st_s.at[pl.ds(f, 16)], jvec, mask=msk)
            plsc.store_compressed(st_d.at[pl.ds(f, 16)], rowvec, mask=msk)
            cnt = plsc.all_reduce_population_count(msk)[0]
            return jnp.minimum(f + cnt, CAPW)

        def group_body(g2, f):
            f = one(g2 * 4, f)
            f = one(g2 * 4 + 1, f)
            f = one(g2 * 4 + 2, f)
            return one(g2 * 4 + 3, f)

        return lax.fori_loop(0, NP // 64, group_body, fill)

    def fetch(row, mbuf):
        return pltpu.async_copy(mask_hbm.at[pl.ds(row * NP, NP)], mbuf, sem_m)

    fetch(rbase, mb0)

    def pair_body(rr, fill):
        r0 = rbase + rr * 2
        pltpu.make_async_copy(mask_hbm.at[pl.ds(r0 * NP, NP)], mb0, sem_m).wait()
        fetch(r0 + 1, mb1)
        fill = scan_row(mb0, r0, fill)
        pltpu.make_async_copy(mask_hbm.at[pl.ds((r0 + 1) * NP, NP)], mb1,
                              sem_m).wait()
        fetch(r0 + 2, mb0)
        return scan_row(mb1, r0 + 1, fill)

    fill = lax.fori_loop(0, (RPW - 1) // 2, pair_body, jnp.int32(0))
    rlast = rbase + RPW - 1
    pltpu.make_async_copy(mask_hbm.at[pl.ds(rlast * NP, NP)], mb0, sem_m).wait()
    fill = scan_row(mb0, rlast, fill)
    pltpu.sync_copy(st_s.at[pl.ds(0, CAPW)], src_out.at[wid])
    pltpu.sync_copy(st_d.at[pl.ds(0, CAPW)], dst_out.at[wid])
    cbuf[...] = jnp.full((16,), 1, jnp.int32) * fill
    pltpu.sync_copy(cbuf, cnt_out.at[wid])


@functools.partial(
    pl.kernel,
    out_type=[
        jax.ShapeDtypeStruct((E_SLAB, TW), jnp.float32),
        jax.ShapeDtypeStruct((E_SLAB, TW), jnp.float32),
    ],
    mesh=_mesh,
    scratch_types=[
        pltpu.VMEM((4, 128), jnp.int32),
        pltpu.VMEM((4, 128), jnp.int32),
        pltpu.VMEM((SG, TW), jnp.float32),
        pltpu.VMEM((SG, TW), jnp.float32),
        pltpu.VMEM((16,), jnp.int32),
        pltpu.VMEM_SHARED((N_TAB, TW), jnp.float32),
        pltpu.SemaphoreType.DMA,
        pltpu.SemaphoreType.DMA,
        pltpu.SemaphoreType.DMA,
    ],
    compiler_params=pltpu.CompilerParams(use_tc_tiling_on_sc=False),
)
def _sc_gather(tab_hbm, src2d, dst2d, cnt_hbm,
               gd_hbm, gs_hbm,
               sidx, didx, gd_b, gs_b, ccv, sh_tab,
               sem_i, sem_g, sem_w):
    cid = lax.axis_index("c")
    sid = lax.axis_index("s")
    wid = sid * NC + cid
    pltpu.sync_copy(cnt_hbm.at[wid], ccv)
    trows = N_TAB // NS
    tb = sid * trows
    pltpu.sync_copy(tab_hbm.at[pl.ds(tb, trows)], sh_tab.at[pl.ds(tb, trows)])
    plsc.subcore_barrier()

    def body(g, carry):
        ebase = wid * EPW + g * SG
        rbase = wid * (EPW // 128) + g * (SG // 128)
        h1 = pltpu.async_copy(src2d.at[pl.ds(rbase, 4)], sidx, sem_i)
        h2 = pltpu.async_copy(dst2d.at[pl.ds(rbase, 4)], didx, sem_i)
        h1.wait()
        h2.wait()
        hs = []
        for j in range(4):
            sl = pl.ds(j * 128, 128)
            hs.append(pltpu.async_copy(sh_tab.at[sidx.at[j]], gs_b.at[sl], sem_g))
            hs.append(pltpu.async_copy(sh_tab.at[didx.at[j]], gd_b.at[sl], sem_g))
        for hh in hs:
            hh.wait()
        w = [pltpu.async_copy(gd_b, gd_hbm.at[pl.ds(ebase, SG)], sem_w),
             pltpu.async_copy(gs_b, gs_hbm.at[pl.ds(ebase, SG)], sem_w)]
        for hh in w:
            hh.wait()
        return carry

    nch = (ccv[...][0] + (SG - 1)) // SG
    lax.fori_loop(0, nch, body, 0)


@functools.partial(
    pl.kernel,
    out_type=[
        jax.ShapeDtypeStruct((NC, N_ACC, F), jnp.float32),
        jax.ShapeDtypeStruct((NC, N_ACC, CW), jnp.float32),
    ],
    mesh=_mesh,
    scratch_types=[
        pltpu.VMEM((4, 128), jnp.int32),
        pltpu.VMEM((SG, F), jnp.float32),
        pltpu.VMEM((SG, CW), jnp.float32),
        pltpu.VMEM((16,), jnp.int32),
        pltpu.VMEM_SHARED((N_ACC, F), jnp.float32),
        pltpu.VMEM_SHARED((N_ACC, CW), jnp.float32),
        pltpu.SemaphoreType.DMA,
        pltpu.SemaphoreType.DMA,
    ],
    compiler_params=pltpu.CompilerParams(use_tc_tiling_on_sc=False),
)
def _sc_scatter(m_hbm, wrel_hbm, dstm2d, cnt_hbm, zh_hbm, zc_hbm,
                out_h, out_c,
                didx, m_b, w_b, ccv, acc_h, acc_c, sem_i, sem_s):
    cid = lax.axis_index("c")
    sid = lax.axis_index("s")
    wid = sid * NC + cid
    pltpu.sync_copy(cnt_hbm.at[wid], ccv)
    rows_per_sub = N_ACC // NS
    rb = sid * rows_per_sub
    pltpu.sync_copy(zh_hbm.at[pl.ds(rb, rows_per_sub)],
                    acc_h.at[pl.ds(rb, rows_per_sub)])
    pltpu.sync_copy(zc_hbm.at[pl.ds(rb, rows_per_sub)],
                    acc_c.at[pl.ds(rb, rows_per_sub)])
    plsc.subcore_barrier()

    def body(g, carry):
        ebase = wid * EPW + g * SG
        rbase = wid * (EPW // 128) + g * (SG // 128)
        pltpu.async_copy(dstm2d.at[pl.ds(rbase, 4)], didx, sem_i).wait()
        a = pltpu.async_copy(m_hbm.at[pl.ds(ebase, SG)], m_b, sem_i)
        b = pltpu.async_copy(wrel_hbm.at[pl.ds(ebase, SG)], w_b, sem_i)
        a.wait()
        b.wait()
        hs = []
        for j in range(4):
            sl = pl.ds(j * 128, 128)
            hs.append(pltpu.async_copy(m_b.at[sl], acc_h.at[didx.at[j]],
                                       sem_s, add=True))
            hs.append(pltpu.async_copy(w_b.at[sl], acc_c.at[didx.at[j]],
                                       sem_s, add=True))
        for hh in hs:
            hh.wait()
        return carry

    nch = (ccv[...][0] + (SG - 1)) // SG
    lax.fori_loop(0, nch, body, 0)
    plsc.subcore_barrier()
    pltpu.sync_copy(acc_h.at[pl.ds(rb, rows_per_sub)],
                    out_h.at[cid].at[pl.ds(rb, rows_per_sub)])
    pltpu.sync_copy(acc_c.at[pl.ds(rb, rows_per_sub)],
                    out_c.at[cid].at[pl.ds(rb, rows_per_sub)])


BPW = CAPW // EDGE_BLK


def _edge_mlp_body(cnt_ref, gd_ref, gs_ref,
                   wi_ref, wj_ref, wr_ref, b1_ref,
                   w2_ref, b2_ref, cw1_ref, cb1_ref, cw2_ref, cb2_ref,
                   cns_ref,
                   m_ref, wrel_ref):
    wkr = pl.program_id(0)
    j = pl.program_id(1)
    jm = jnp.maximum((cnt_ref[wkr] + EDGE_BLK - 1) // EDGE_BLK - 1, 0)

    @pl.when(j <= jm)
    def _():
        gd = gd_ref[...]
        gs = gs_ref[...]
        xi = gd[:, :F]
        xj = gs[:, :F]
        rel = jnp.pad(gs[:, F:TW] - gd[:, F:TW], ((0, 0), (0, CW - (TW - F))))
        rd = jnp.sum(rel * rel, axis=-1, keepdims=True)
        pre = (jnp.dot(xi, wi_ref[...], preferred_element_type=jnp.float32)
               + jnp.dot(xj, wj_ref[...], preferred_element_type=jnp.float32)
               + rd * wr_ref[...] + b1_ref[...])
        m1 = _silu(pre)
        m = _silu(jnp.dot(m1, w2_ref[...], preferred_element_type=jnp.float32)
                  + b2_ref[...])
        t = _silu(jnp.dot(m, cw1_ref[...], preferred_element_type=jnp.float32)
                  + cb1_ref[...])
        w = (jnp.dot(t, cw2_ref[...], preferred_element_type=jnp.float32)
             + cb2_ref[...])
        nrm = jnp.sqrt(rd)
        rel_n = rel / jnp.maximum(nrm, 1e-8) * cns_ref[0, 0]
        m_ref[...] = m
        wrel_ref[...] = w * rel_n


def _edge_mlp(gd, gs, cnts, wi, wj, wr, b1, w2, b2, cw1, cb1, cw2, cb2, cns):
    def eb(w, j, cref):
        jm = jnp.maximum((cref[w] + EDGE_BLK - 1) // EDGE_BLK - 1, 0)
        return (w * BPW + jnp.minimum(j, jm), 0)

    full = lambda w, j, cref: (0, 0)
    grid_spec = pltpu.PrefetchScalarGridSpec(
        num_scalar_prefetch=1,
        grid=(NW, BPW),
        in_specs=[
            pl.BlockSpec((EDGE_BLK, TW), eb),
            pl.BlockSpec((EDGE_BLK, TW), eb),
            pl.BlockSpec((F, 2 * EIN), full),
            pl.BlockSpec((F, 2 * EIN), full),
            pl.BlockSpec((1, 2 * EIN), full),
            pl.BlockSpec((1, 2 * EIN), full),
            pl.BlockSpec((2 * EIN, M), full),
            pl.BlockSpec((1, M), full),
            pl.BlockSpec((M, 4 * M), full),
            pl.BlockSpec((1, 4 * M), full),
            pl.BlockSpec((4 * M, 1), full),
            pl.BlockSpec((1, 1), full),
            pl.BlockSpec((1, 1), full),
        ],
        out_specs=[
            pl.BlockSpec((EDGE_BLK, M), eb),
            pl.BlockSpec((EDGE_BLK, CW), eb),
        ],
    )
    return pl.pallas_call(
        _edge_mlp_body,
        grid_spec=grid_spec,
        out_shape=[
            jax.ShapeDtypeStruct((E_SLAB, M), jnp.float32),
            jax.ShapeDtypeStruct((E_SLAB, CW), jnp.float32),
        ],
    )(cnts, gd, gs, wi, wj, wr, b1, w2, b2, cw1, cb1, cw2, cb2, cns)


def kernel(coors, feats, emb_w, emb_b, ew1, eb1, ew2, eb2, nnw, nnb,
           nw1, nb1, nw2, nb2, cw1, cb1, cw2, cb2, cns, ln_w, ln_b):
    n = coors.shape[0]
    c8 = jnp.zeros((NP, 8), jnp.float32).at[:n, :3].set(coors)
    r2 = jnp.full((NP,), 1e9, jnp.float32).at[:n].set(
        jnp.sum(coors * coors, axis=1))
    mask = _mask_kernel(c8, c8.T, r2[:, None], r2[None, :])
    src_slab, dst_slab, cnt = _sc_compact(mask.reshape(NP * NP))
    src2d = src_slab.reshape(E_SLAB // 128, 128)
    dst2d = dst_slab.reshape(E_SLAB // 128, 128)

    zh = jnp.zeros((N_ACC, F), jnp.float32)
    zc = jnp.zeros((N_ACC, CW), jnp.float32)

    h = feats @ emb_w + emb_b
    c = coors
    for l in range(L):
        tab = jnp.zeros((N_TAB, TW), jnp.float32)
        tab = tab.at[:n, :F].set(h).at[:n, F:F + 3].set(c)
        gd, gs = _sc_gather(tab, src2d, dst2d, cnt)
        m, wrel = _edge_mlp(
            gd, gs, cnt[:, 0],
            ew1[l][:F], ew1[l][F:2 * F], ew1[l][2 * F:2 * F + 1],
            eb1[l][None, :], ew2[l], eb2[l][None, :],
            cw1[l], cb1[l][None, :], cw2[l], cb2[l][None, :],
            cns[l][None, :],
        )
        hacc, cacc = _sc_scatter(m, wrel, dst2d, cnt, zh, zc)
        m_i = hacc[0, :N] + hacc[1, :N]
        c = c + (cacc[0, :N, :3] + cacc[1, :N, :3])
        mu = jnp.mean(h)
        sd = jnp.std(h)
        hf = (h - mu) / (sd + 1e-5) * nnw[l] + nnb[l]
        ho = _silu(jnp.concatenate([hf, m_i], axis=-1) @ nw1[l] + nb1[l]) @ nw2[l] + nb2[l]
        z = 2.0 * h + ho
        zm = jnp.mean(z, axis=-1, keepdims=True)
        zv = jnp.var(z, axis=-1, keepdims=True)
        h = (z - zm) / jnp.sqrt(zv + 1e-5) * ln_w + ln_b
    return c, h
